# baseline (device time: 219363 ns/iter reference)
import jax
import jax.numpy as jnp
from jax import lax
from jax.experimental import pallas as pl
from jax.experimental.pallas import tpu as pltpu

N_DEV = 32
BLK = 64
GRP = 8
N_GRP = N_DEV // GRP


def kernel(x, Wq, K_ext, V_ext, Wo):
    B, S, DM = x.shape
    Hq, Dh = K_ext.shape[2], K_ext.shape[3]
    HD = Hq * Dh
    SP = S + Hq

    K2 = K_ext.reshape(B, S, HD)
    V2 = V_ext.reshape(B, S, HD)

    def body(x_ref, wq_ref, k_ref, v_ref, wo_ref, out_ref,
             q_buf, p_in, p_out,
             q_send_sems, q_recv_sems, p_send_sems, p_recv_sems):
        my = lax.axis_index("i")
        bf = jnp.bfloat16

        wq = wq_ref[...].astype(bf)
        wo = wo_ref[...].astype(bf)

        for b in range(B):
            qb = jnp.dot(x_ref[b].astype(bf), wq,
                         preferred_element_type=jnp.float32) * 0.125
            q_buf[pl.ds(my, 1), b] = qb.astype(bf)[None]

        for jj in range(N_DEV - 1):
            @pl.when(jj < my)
            def _():
                rdma = pltpu.make_async_remote_copy(
                    src_ref=q_buf.at[my],
                    dst_ref=q_buf.at[my],
                    send_sem=q_send_sems.at[jj],
                    recv_sem=q_recv_sems.at[my],
                    device_id=(jj,),
                    device_id_type=pl.DeviceIdType.MESH,
                )
                rdma.start()

        kl = [k_ref[b].astype(bf) for b in range(B)]
        vl = [v_ref[b].astype(bf) for b in range(B)]

        rb = lax.broadcasted_iota(jnp.int32, (S, S), 0) // BLK
        cb = lax.broadcasted_iota(jnp.int32, (S, S), 1) // BLK
        mask0 = cb <= rb

        ctx0 = []
        s0 = []
        for b in range(B):
            ctx_h = []
            s_h = []
            for h in range(Hq):
                hs = slice(h * Dh, (h + 1) * Dh)
                q = q_buf[pl.ds(my, 1), b, :, hs][0]
                sc = lax.dot_general(q, kl[b][:, hs],
                                     (((1,), (1,)), ((), ())),
                                     preferred_element_type=jnp.float32)
                e = jnp.where(mask0, jnp.exp(sc), 0.0)
                s_h.append(jnp.sum(e, axis=1))
                ctx_h.append(jnp.dot(e.astype(bf), vl[b][:, hs],
                                     preferred_element_type=jnp.float32))
            ctx0.append(jnp.concatenate(ctx_h, axis=1))
            s0.append(jnp.stack(s_h, axis=0))
        ctx_acc = jnp.stack(ctx0, axis=0)
        s_acc = jnp.stack(s0, axis=0)

        for g in range(N_GRP):
            lo, hi = g * GRP, (g + 1) * GRP
            for i in range(lo, hi):
                @pl.when(i > my)
                def _():
                    recv = pltpu.make_async_remote_copy(
                        src_ref=q_buf.at[i], dst_ref=q_buf.at[i],
                        send_sem=q_send_sems.at[0],
                        recv_sem=q_recv_sems.at[i],
                        device_id=(i,),
                        device_id_type=pl.DeviceIdType.MESH,
                    )
                    recv.wait_recv()

            @pl.when(my < hi - 1)
            def _():
                for b in range(B):
                    for h in range(Hq):
                        hs = slice(h * Dh, (h + 1) * Dh)
                        qg = q_buf[lo:hi, b, :, hs].reshape(GRP * S, Dh)
                        sc = lax.dot_general(
                            qg, kl[b][:, hs], (((1,), (1,)), ((), ())),
                            preferred_element_type=jnp.float32)
                        e = jnp.exp(sc)
                        p_out[lo:hi, b, S + h, :] = (
                            jnp.sum(e, axis=1).astype(bf).reshape(GRP, S))
                        p_out[lo:hi, b, :S, hs] = (
                            jnp.dot(e.astype(bf), vl[b][:, hs],
                                    preferred_element_type=jnp.float32)
                            .astype(bf).reshape(GRP, S, Dh))

            for i in range(lo, hi):
                @pl.when(i > my)
                def _():
                    send = pltpu.make_async_remote_copy(
                        src_ref=p_out.at[i],
                        dst_ref=p_in.at[my],
                        send_sem=p_send_sems.at[i],
                        recv_sem=p_recv_sems.at[my],
                        device_id=(i,),
                        device_id_type=pl.DeviceIdType.MESH,
                    )
                    send.start()

        for j in range(N_DEV - 1):
            @pl.when(j < my)
            def _():
                recv = pltpu.make_async_remote_copy(
                    src_ref=p_in.at[j], dst_ref=p_in.at[j],
                    send_sem=p_send_sems.at[0],
                    recv_sem=p_recv_sems.at[j],
                    device_id=(j,), device_id_type=pl.DeviceIdType.MESH,
                )
                recv.wait_recv()

        keep = lax.broadcasted_iota(jnp.int32, (N_DEV, 1, 1, 1), 0) < my
        red = jnp.sum(
            jnp.where(keep, p_in[...].astype(jnp.float32), 0.0), axis=0
        )
        ctx_acc = ctx_acc + red[:, :S, :]
        s_acc = s_acc + red[:, S:, :]

        for b in range(B):
            cols = []
            for h in range(Hq):
                hs = slice(h * Dh, (h + 1) * Dh)
                cols.append(ctx_acc[b][:, hs] / s_acc[b, h][:, None])
            ctx = jnp.concatenate(cols, axis=1).astype(bf)
            out_ref[b] = jnp.dot(ctx, wo,
                                 preferred_element_type=jnp.float32)

        for jj in range(N_DEV - 1):
            @pl.when(jj < my)
            def _():
                d = pltpu.make_async_remote_copy(
                    src_ref=q_buf.at[my], dst_ref=q_buf.at[my],
                    send_sem=q_send_sems.at[jj],
                    recv_sem=q_recv_sems.at[0],
                    device_id=(jj,), device_id_type=pl.DeviceIdType.MESH,
                )
                d.wait_send()
        for i in range(N_DEV):
            @pl.when(i > my)
            def _():
                d = pltpu.make_async_remote_copy(
                    src_ref=p_out.at[i], dst_ref=p_in.at[my],
                    send_sem=p_send_sems.at[i],
                    recv_sem=p_recv_sems.at[0],
                    device_id=(i,), device_id_type=pl.DeviceIdType.MESH,
                )
                d.wait_send()

    return pl.pallas_call(
        body,
        out_shape=jax.ShapeDtypeStruct((B, S, DM), jnp.float32),
        in_specs=[pl.BlockSpec(memory_space=pltpu.VMEM)] * 5,
        out_specs=pl.BlockSpec(memory_space=pltpu.VMEM),
        scratch_shapes=[
            pltpu.VMEM((N_DEV, B, S, HD), jnp.bfloat16),
            pltpu.VMEM((N_DEV, B, SP, HD), jnp.bfloat16),
            pltpu.VMEM((N_DEV, B, SP, HD), jnp.bfloat16),
            pltpu.SemaphoreType.DMA((N_DEV - 1,)),
            pltpu.SemaphoreType.DMA((N_DEV,)),
            pltpu.SemaphoreType.DMA((N_DEV,)),
            pltpu.SemaphoreType.DMA((N_DEV,)),
        ],
        compiler_params=pltpu.CompilerParams(
            vmem_limit_bytes=100 * 1024 * 1024,
        ),
    )(x, Wq, K2, V2, Wo)
